# baseline (speedup 1.0000x reference)
import jax
import jax.numpy as jnp
from jax.experimental import pallas as pl

SPIKE_THRESHOLD = 0.01
_BLK = 4096


def _fused_block(x_ref, wt_ref, b_ref, o_ref):
    xb = x_ref[...]
    y = jnp.dot(xb, wt_ref[...], preferred_element_type=jnp.float32) + b_ref[...]
    spike = (jnp.abs(xb) > SPIKE_THRESHOLD).any(axis=1, keepdims=True)
    o_ref[...] = jnp.where(spike, y, 0.0)


def kernel(x, W, b):
    B, T, S, D = x.shape
    n = B * T * S
    xf = x.reshape(n, D)
    out = pl.pallas_call(
        _fused_block,
        grid=(n // _BLK,),
        in_specs=[
            pl.BlockSpec((_BLK, D), lambda i: (i, 0)),
            pl.BlockSpec((D, D), lambda i: (0, 0)),
            pl.BlockSpec((1, D), lambda i: (0, 0)),
        ],
        out_specs=pl.BlockSpec((_BLK, D), lambda i: (i, 0)),
        out_shape=jax.ShapeDtypeStruct((n, D), x.dtype),
    )(xf, W.T, b.reshape(1, D))
    return out.reshape(B, T, S, D)
